# Initial kernel scaffold; baseline (speedup 1.0000x reference)
import jax
import jax.numpy as jnp
from jax.experimental import pallas as pl


def kernel(grammian):
    raise NotImplementedError("write your pallas kernel here")


# speedup vs baseline: 12.3108x; 12.3108x over previous
import functools

import jax
import jax.numpy as jnp
from jax import lax
from jax.experimental import pallas as pl
from jax.experimental.pallas import tpu as pltpu
from jax.experimental.pallas import tpu_sc as plsc

N = 4096
L = 16
NC = 2
NS = 16
NW = NC * NS
ROWS_PER_W = N // NW
NCHUNK = N // L
U = 4
NGRP = N // (L * U)
EPS = 1e-8
IMAX = 2**31 - 1


def _diag_body(g_ref, d_ref):
    blk = g_ref[...]
    rio = lax.broadcasted_iota(jnp.int32, (128, 128), 0)
    cio = lax.broadcasted_iota(jnp.int32, (128, 128), 1)
    d_ref[0, :] = jnp.sum(jnp.where(rio == cio, blk, 0.0), axis=0)


def _extract_diag(g):
    out = pl.pallas_call(
        _diag_body,
        grid=(N // 128,),
        in_specs=[pl.BlockSpec((128, 128), lambda r: (r, r))],
        out_specs=pl.BlockSpec((1, 128), lambda r: (0, r)),
        out_shape=jax.ShapeDtypeStruct((1, N), jnp.float32),
    )(g)
    return out.reshape(N)


def _sc_body(gram, diag_hbm, part, diag_v, rowbuf, stage):
    c = lax.axis_index("c")
    s = lax.axis_index("s")
    wid = s * NC + c

    pltpu.sync_copy(diag_hbm, diag_v.at[pl.ds(0, N)])
    io16 = lax.iota(jnp.int32, L)

    def row_step(t, carry):
        i = wid + NW * t
        pltpu.sync_copy(gram.at[i], rowbuf)
        i_vec = jnp.full((L,), i, jnp.int32)
        di = jnp.full((L,), diag_v[pl.ds(i, L)][0], jnp.float32)

        def chunk(jb, idx, cc, jvec=None):
            mv, mi, mg = cc
            v = rowbuf[pl.ds(jb, L)]
            dj = diag_v[pl.ds(jb, L)]
            t1 = dj - v
            den = di + dj - (v + v) + EPS
            g = t1 / den
            cond1 = v < dj
            cond2 = v < di
            g = jnp.where(cond1, g, 0.0)
            g = jnp.where(cond2, g, 1.0)
            cost = dj + g * (v - dj)
            cost = jnp.where(cond1, cost, dj)
            cost = jnp.where(cond2, cost, di)
            if jvec is not None:
                cost = jnp.where(jvec > i_vec, cost, jnp.inf)
            better = cost < mv
            return (jnp.where(better, cost, mv),
                    jnp.where(better, idx, mi),
                    jnp.where(better, g, mg))

        g0 = i // (L * U)
        jb0 = g0 * (L * U)
        cc = carry
        for u in range(U):
            jvec = io16 + (jb0 + u * L)
            cc = chunk(jb0 + u * L, i_vec * N + jvec, cc, jvec=jvec)

        idx0 = i_vec * N + io16 + (jb0 + L * U)

        def group(gg, st):
            idxv, gcc = st
            jb = gg * (L * U)
            for u in range(U):
                gcc = chunk(jb + u * L, idxv + u * L, gcc)
            return idxv + L * U, gcc

        _, cc = lax.fori_loop(g0 + 1, NGRP, group, (idx0, cc))
        return cc

    init = (jnp.full((L,), jnp.inf, jnp.float32),
            jnp.zeros((L,), jnp.int32),
            jnp.zeros((L,), jnp.float32))
    mval, midx, mgam = lax.fori_loop(0, ROWS_PER_W, row_step, init)

    stage[pl.ds(0, L)] = mval
    stage[pl.ds(L, L)] = lax.bitcast_convert_type(midx, jnp.float32)
    stage[pl.ds(2 * L, L)] = mgam
    pltpu.sync_copy(stage, part.at[wid])


def _sc_solve(gram, diag):
    mesh = plsc.VectorSubcoreMesh(core_axis_name="c", subcore_axis_name="s")
    run = functools.partial(
        pl.kernel,
        mesh=mesh,
        out_type=jax.ShapeDtypeStruct((NW, 3 * L), jnp.float32),
        scratch_types=[
            pltpu.VMEM((N + L,), jnp.float32),
            pltpu.VMEM((N,), jnp.float32),
            pltpu.VMEM((3 * L,), jnp.float32),
        ],
    )(_sc_body)
    return run(gram, diag)


def _merge_body(p_ref, o_ref):
    p = p_ref[...]
    vals = p[:, 0:L]
    idx = lax.bitcast_convert_type(p[:, L:2 * L], jnp.int32)
    gams = p[:, 2 * L:3 * L]
    m = jnp.min(vals)
    tied = vals == m
    bid = jnp.min(jnp.where(tied, idx, IMAX))
    g = jnp.sum(jnp.where(tied & (idx == bid), gams, 0.0))
    i_w = lax.shift_right_logical(bid, 12)
    j_w = jnp.bitwise_and(bid, N - 1)
    cio = lax.broadcasted_iota(jnp.int32, (1, N), 1)
    o_ref[...] = jnp.where(cio == i_w, g,
                 jnp.where(cio == j_w, 1.0 - g, 0.0))


def _merge(partials):
    out = pl.pallas_call(
        _merge_body,
        out_shape=jax.ShapeDtypeStruct((1, N), jnp.float32),
    )(partials)
    return out.reshape(N)


def kernel(grammian):
    diag = _extract_diag(grammian)
    partials = _sc_solve(grammian, diag)
    return _merge(partials)
